# baseline (speedup 1.0000x reference)
import functools

import jax
import jax.numpy as jnp
from jax import lax
from jax.experimental import pallas as pl
from jax.experimental.pallas import tpu as pltpu
from jax.experimental.pallas import tpu_sc as plsc

N_NODES = 10000
N_EDGES = 320000
D_FEAT = 128
HIDDEN = 64
N_REL = 4

NC = 2
NS = 16
NW = NC * NS
L = 16

EPT = N_EDGES // NW
CHUNK = 128
NCHUNK = N_EDGES // CHUNK
N_PAD = 10240
ROWS_PER_TILE = N_PAD // NS


def _chunk_start(w):
    return (NCHUNK * w // NW) // 8 * 8


_COUNTS = sorted({
    (NCHUNK if w == NW - 1 else _chunk_start(w + 1)) - _chunk_start(w)
    for w in range(NW)
})
MAXC = max(_COUNTS)
NBUF = 4
assert all(c % NBUF == 0 for c in _COUNTS)

@functools.lru_cache(maxsize=None)
def _make_degrees_and_gidx():
    mesh = plsc.VectorSubcoreMesh(
        core_axis_name="c", subcore_axis_name="s", num_cores=NC, num_subcores=NS
    )
    return functools.partial(
        pl.kernel,
        out_type=(
            jax.ShapeDtypeStruct((NW, 1, N_NODES), jnp.float32),
            jax.ShapeDtypeStruct((NW, 1, N_NODES), jnp.float32),
            jax.ShapeDtypeStruct((N_EDGES,), jnp.int32),
        ),
        mesh=mesh,
        compiler_params=pltpu.CompilerParams(
            needs_layout_passes=False, use_tc_tiling_on_sc=False
        ),
        scratch_types=[
            pltpu.VMEM((N_NODES,), jnp.float32),
            pltpu.VMEM((N_NODES,), jnp.float32),
            pltpu.VMEM((EPT,), jnp.int32),
            pltpu.VMEM((EPT,), jnp.int32),
            pltpu.VMEM((EPT,), jnp.int32),
            pltpu.VMEM((EPT,), jnp.int32),
        ],
    )(_degrees_and_gidx_body)


def _degrees_and_gidx_body(sen_hbm, rcv_hbm, et_hbm, hs_out, hr_out, gx_out,
                           hs, hr, sen, rcv, et, gix):
    cid = lax.axis_index("c")
    sid = lax.axis_index("s")
    wid = sid * NC + cid
    base = wid * EPT

    def zero_body(j, _):
        z = jnp.zeros((L,), jnp.float32)
        hs[pl.ds(j * L, L)] = z
        hr[pl.ds(j * L, L)] = z
        return 0

    lax.fori_loop(0, N_NODES // L, zero_body, 0)

    pltpu.sync_copy(sen_hbm.at[pl.ds(base, EPT)], sen)
    pltpu.sync_copy(rcv_hbm.at[pl.ds(base, EPT)], rcv)
    pltpu.sync_copy(et_hbm.at[pl.ds(base, EPT)], et)

    ones = jnp.ones((L,), jnp.float32)

    def body(k, _):
        s_v = sen[pl.ds(k * L, L)]
        r_v = rcv[pl.ds(k * L, L)]
        t_v = et[pl.ds(k * L, L)]
        plsc.addupdate_scatter(hs, [s_v], ones)
        plsc.addupdate_scatter(hr, [r_v], ones)
        gix[pl.ds(k * L, L)] = t_v * N_NODES + s_v
        return 0

    lax.fori_loop(0, EPT // L, body, 0)

    pltpu.sync_copy(gix, gx_out.at[pl.ds(base, EPT)])
    pltpu.sync_copy(hs, hs_out.at[wid, 0])
    pltpu.sync_copy(hr, hr_out.at[wid, 0])


BN = 1000


def _rel_matmul_body(nodes_ref, k_ref, out_ref):
    out_ref[...] = jnp.dot(nodes_ref[...], k_ref[0],
                           preferred_element_type=jnp.float32)


_rel_matmul = pl.pallas_call(
    _rel_matmul_body,
    grid=(N_REL,),
    in_specs=[
        pl.BlockSpec((N_NODES, D_FEAT), lambda r: (0, 0)),
        pl.BlockSpec((1, D_FEAT, HIDDEN), lambda r: (r, 0, 0)),
    ],
    out_specs=pl.BlockSpec((N_NODES, HIDDEN), lambda r: (r, 0)),
    out_shape=jax.ShapeDtypeStruct((N_REL * N_NODES, HIDDEN), jnp.float32),
)


def _scale_tab_body(hs_ref, a_ref, out_ref):
    sdeg_col = lax.dot_general(
        hs_ref[...], jnp.ones((NW, 1), jnp.float32),
        (((0,), (0,)), ((), ())), preferred_element_type=jnp.float32)
    ns_col = lax.rsqrt(jnp.maximum(sdeg_col, 1.0))
    out_ref[...] = a_ref[...] * ns_col


_scale_tab = pl.pallas_call(
    _scale_tab_body,
    grid=(N_REL,),
    in_specs=[
        pl.BlockSpec((NW, N_NODES), lambda r: (0, 0)),
        pl.BlockSpec((N_NODES, HIDDEN), lambda r: (r, 0)),
    ],
    out_specs=pl.BlockSpec((N_NODES, HIDDEN), lambda r: (r, 0)),
    out_shape=jax.ShapeDtypeStruct((N_REL * N_NODES, HIDDEN), jnp.float32),
)


@functools.lru_cache(maxsize=None)
def _make_edge_aggregate():
    mesh = plsc.VectorSubcoreMesh(
        core_axis_name="c", subcore_axis_name="s", num_cores=NC, num_subcores=NS
    )
    return functools.partial(
        pl.kernel,
        out_type=jax.ShapeDtypeStruct((NC, N_PAD, HIDDEN), jnp.float32),
        mesh=mesh,
        compiler_params=pltpu.CompilerParams(
            needs_layout_passes=False, use_tc_tiling_on_sc=False
        ),
        scratch_types=[
            pltpu.VMEM_SHARED((N_PAD, HIDDEN), jnp.float32),
            pltpu.VMEM((MAXC, CHUNK), jnp.int32),
            pltpu.VMEM((MAXC, CHUNK), jnp.int32),
        ]
        + [pltpu.VMEM((CHUNK, HIDDEN), jnp.float32) for _ in range(NBUF)]
        + [pltpu.SemaphoreType.DMA for _ in range(2 * NBUF)],
    )(_edge_aggregate_body)


def _edge_aggregate_body(a_hbm, gx_hbm, rc_hbm, z_hbm, out_hbm,
                         agg_sh, gix, rcv, *rest):
    bufs = rest[:NBUF]
    gsems = rest[NBUF:2 * NBUF]
    ssems = rest[2 * NBUF:]
    cid = lax.axis_index("c")
    sid = lax.axis_index("s")
    wid = sid * NC + cid

    pltpu.sync_copy(z_hbm, bufs[0])
    for j in range(ROWS_PER_TILE // CHUNK):
        pltpu.sync_copy(
            bufs[0], agg_sh.at[pl.ds(sid * ROWS_PER_TILE + j * CHUNK, CHUNK)])

    start = _chunk_start(wid)
    cnt = jnp.where(wid == NW - 1, NCHUNK, _chunk_start(wid + 1)) - start
    for c in _COUNTS:
        @pl.when(cnt == c)
        def _(c=c):
            pltpu.sync_copy(gx_hbm.at[pl.ds(start, c)], gix.at[pl.ds(0, c)])
            pltpu.sync_copy(rc_hbm.at[pl.ds(start, c)], rcv.at[pl.ds(0, c)])

    plsc.subcore_barrier()

    for b in range(NBUF):
        pltpu.async_copy(a_hbm.at[gix.at[b]], bufs[b], gsems[b])

    def quad(i, _):
        k0 = NBUF * i
        for b in range(NBUF):
            pltpu.make_async_copy(a_hbm.at[gix.at[k0 + b]], bufs[b], gsems[b]).wait()
            pltpu.async_copy(bufs[b], agg_sh.at[rcv.at[k0 + b]], ssems[b], add=True)
        for b in range(NBUF):
            @pl.when(k0 + b + NBUF < cnt)
            def _(b=b):
                pltpu.make_async_copy(bufs[b], agg_sh.at[rcv.at[k0]], ssems[b]).wait()
                pltpu.async_copy(a_hbm.at[gix.at[k0 + b + NBUF]], bufs[b], gsems[b])
        return 0

    lax.fori_loop(0, cnt // NBUF, quad, 0)

    for b in range(NBUF):
        pltpu.make_async_copy(bufs[b], agg_sh.at[rcv.at[0]], ssems[b]).wait()

    plsc.subcore_barrier()

    for j in range(ROWS_PER_TILE // CHUNK):
        r0 = sid * ROWS_PER_TILE + j * CHUNK
        pltpu.sync_copy(agg_sh.at[pl.ds(r0, CHUNK)], bufs[0])
        pltpu.sync_copy(bufs[0], out_hbm.at[cid, pl.ds(r0, CHUNK)])


def _finalize_body(parts_ref, hr_ref, nn_ref, w_ref, b_ref, out_ref):
    rdeg = jnp.sum(hr_ref[...], axis=0)
    nr = lax.rsqrt(jnp.maximum(rdeg, 1.0))
    nr_row = jnp.concatenate(
        [nr, jnp.zeros((N_PAD - N_NODES,), jnp.float32)]
    ).reshape(1, N_PAD)
    h = jnp.maximum(parts_ref[0] + parts_ref[1], 0.0)
    pooled = jnp.dot(nr_row, h, preferred_element_type=jnp.float32)
    cnt = jnp.maximum(nn_ref[0].astype(jnp.float32), 1.0)
    emb = pooled / cnt
    out_ref[...] = jnp.dot(emb, w_ref[...], preferred_element_type=jnp.float32) + b_ref[...]


_finalize = pl.pallas_call(
    _finalize_body,
    in_specs=[
        pl.BlockSpec(memory_space=pltpu.VMEM),
        pl.BlockSpec(memory_space=pltpu.VMEM),
        pl.BlockSpec(memory_space=pltpu.SMEM),
        pl.BlockSpec(memory_space=pltpu.VMEM),
        pl.BlockSpec(memory_space=pltpu.VMEM),
    ],
    out_shape=jax.ShapeDtypeStruct((1, HIDDEN), jnp.float32),
)


def kernel(nodes, senders, receivers, edge_types, n_node, kernels_0,
           dense_kernel, dense_bias):
    a_raw = _rel_matmul(nodes, kernels_0)
    hist_s, hist_r, gidx = _make_degrees_and_gidx()(senders, receivers, edge_types)
    a_tab = _scale_tab(hist_s.reshape(NW, N_NODES), a_raw)
    zeros = jnp.zeros((CHUNK, HIDDEN), jnp.float32)
    parts = _make_edge_aggregate()(
        a_tab,
        gidx.reshape(NCHUNK, CHUNK),
        receivers.reshape(NCHUNK, CHUNK),
        zeros,
    )
    out = _finalize(parts, hist_r.reshape(NW, N_NODES), n_node, dense_kernel,
                    dense_bias.reshape(1, HIDDEN))
    return out.reshape(HIDDEN)


# speedup vs baseline: 1.0451x; 1.0451x over previous
import functools

import jax
import jax.numpy as jnp
from jax import lax
from jax.experimental import pallas as pl
from jax.experimental.pallas import tpu as pltpu
from jax.experimental.pallas import tpu_sc as plsc

N_NODES = 10000
N_EDGES = 320000
D_FEAT = 128
HIDDEN = 64
N_REL = 4

NC = 2
NS = 16
NW = NC * NS
L = 16

EPT = N_EDGES // NW
CHUNK = 128
NCHUNK = N_EDGES // CHUNK
N_PAD = 10240
ROWS_PER_TILE = N_PAD // NS


def _chunk_start(w):
    return (NCHUNK * w // NW) // 8 * 8


_COUNTS = sorted({
    (NCHUNK if w == NW - 1 else _chunk_start(w + 1)) - _chunk_start(w)
    for w in range(NW)
})
MAXC = max(_COUNTS)
NBUF = 4
assert all(c % NBUF == 0 for c in _COUNTS)

@functools.lru_cache(maxsize=None)
def _make_degrees_and_gidx():
    mesh = plsc.VectorSubcoreMesh(
        core_axis_name="c", subcore_axis_name="s", num_cores=NC, num_subcores=NS
    )
    return functools.partial(
        pl.kernel,
        out_type=(
            jax.ShapeDtypeStruct((NW, 1, N_NODES), jnp.float32),
            jax.ShapeDtypeStruct((NW, 1, N_NODES), jnp.float32),
            jax.ShapeDtypeStruct((N_EDGES,), jnp.int32),
        ),
        mesh=mesh,
        compiler_params=pltpu.CompilerParams(
            needs_layout_passes=False, use_tc_tiling_on_sc=False
        ),
        scratch_types=[
            pltpu.VMEM((N_NODES,), jnp.float32),
            pltpu.VMEM((N_NODES,), jnp.float32),
            pltpu.VMEM((EPT,), jnp.int32),
            pltpu.VMEM((EPT,), jnp.int32),
            pltpu.VMEM((EPT,), jnp.int32),
            pltpu.VMEM((EPT,), jnp.int32),
            pltpu.SemaphoreType.DMA,
        ],
    )(_degrees_and_gidx_body)


def _degrees_and_gidx_body(sen_hbm, rcv_hbm, et_hbm, hs_out, hr_out, gx_out,
                           hs, hr, sen, rcv, et, gix, ldsem):
    cid = lax.axis_index("c")
    sid = lax.axis_index("s")
    wid = sid * NC + cid
    base = wid * EPT

    pltpu.async_copy(sen_hbm.at[pl.ds(base, EPT)], sen, ldsem)
    pltpu.async_copy(rcv_hbm.at[pl.ds(base, EPT)], rcv, ldsem)
    pltpu.async_copy(et_hbm.at[pl.ds(base, EPT)], et, ldsem)

    def zero_body(j, _):
        z = jnp.zeros((L,), jnp.float32)
        hs[pl.ds(j * L, L)] = z
        hr[pl.ds(j * L, L)] = z
        return 0

    lax.fori_loop(0, N_NODES // L, zero_body, 0)

    pltpu.make_async_copy(sen_hbm.at[pl.ds(base, EPT)], sen, ldsem).wait()
    pltpu.make_async_copy(rcv_hbm.at[pl.ds(base, EPT)], rcv, ldsem).wait()
    pltpu.make_async_copy(et_hbm.at[pl.ds(base, EPT)], et, ldsem).wait()

    ones = jnp.ones((L,), jnp.float32)

    def body(k, _):
        s_v = sen[pl.ds(k * L, L)]
        r_v = rcv[pl.ds(k * L, L)]
        t_v = et[pl.ds(k * L, L)]
        plsc.addupdate_scatter(hs, [s_v], ones)
        plsc.addupdate_scatter(hr, [r_v], ones)
        gix[pl.ds(k * L, L)] = t_v * N_NODES + s_v
        return 0

    lax.fori_loop(0, EPT // L, body, 0)

    pltpu.sync_copy(gix, gx_out.at[pl.ds(base, EPT)])
    pltpu.sync_copy(hs, hs_out.at[wid, 0])
    pltpu.sync_copy(hr, hr_out.at[wid, 0])


BN = 1000


def _rel_matmul_body(hs_ref, nodes_ref, k_ref, out_ref):
    sdeg_col = lax.dot_general(
        hs_ref[...], jnp.ones((NW, 1), jnp.float32),
        (((0,), (0,)), ((), ())), preferred_element_type=jnp.float32)
    ns_col = lax.rsqrt(jnp.maximum(sdeg_col, 1.0))
    x = nodes_ref[...] * ns_col
    out_ref[...] = jnp.dot(x, k_ref[0], preferred_element_type=jnp.float32)


_rel_matmul = pl.pallas_call(
    _rel_matmul_body,
    grid=(N_REL,),
    in_specs=[
        pl.BlockSpec((NW, N_NODES), lambda r: (0, 0)),
        pl.BlockSpec((N_NODES, D_FEAT), lambda r: (0, 0)),
        pl.BlockSpec((1, D_FEAT, HIDDEN), lambda r: (r, 0, 0)),
    ],
    out_specs=pl.BlockSpec((N_NODES, HIDDEN), lambda r: (r, 0)),
    out_shape=jax.ShapeDtypeStruct((N_REL * N_NODES, HIDDEN), jnp.float32),
)


@functools.lru_cache(maxsize=None)
def _make_edge_aggregate():
    mesh = plsc.VectorSubcoreMesh(
        core_axis_name="c", subcore_axis_name="s", num_cores=NC, num_subcores=NS
    )
    return functools.partial(
        pl.kernel,
        out_type=jax.ShapeDtypeStruct((NC, N_PAD, HIDDEN), jnp.float32),
        mesh=mesh,
        compiler_params=pltpu.CompilerParams(
            needs_layout_passes=False, use_tc_tiling_on_sc=False
        ),
        scratch_types=[
            pltpu.VMEM_SHARED((N_PAD, HIDDEN), jnp.float32),
            pltpu.VMEM((MAXC, CHUNK), jnp.int32),
            pltpu.VMEM((MAXC, CHUNK), jnp.int32),
        ]
        + [pltpu.VMEM((CHUNK, HIDDEN), jnp.float32) for _ in range(NBUF)]
        + [pltpu.SemaphoreType.DMA for _ in range(2 * NBUF)],
    )(_edge_aggregate_body)


def _edge_aggregate_body(a_hbm, gx_hbm, rc_hbm, z_hbm, out_hbm,
                         agg_sh, gix, rcv, *rest):
    bufs = rest[:NBUF]
    gsems = rest[NBUF:2 * NBUF]
    ssems = rest[2 * NBUF:]
    cid = lax.axis_index("c")
    sid = lax.axis_index("s")
    wid = sid * NC + cid

    pltpu.sync_copy(z_hbm, agg_sh.at[pl.ds(sid * ROWS_PER_TILE, ROWS_PER_TILE)])

    start = _chunk_start(wid)
    cnt = jnp.where(wid == NW - 1, NCHUNK, _chunk_start(wid + 1)) - start
    for c in _COUNTS:
        @pl.when(cnt == c)
        def _(c=c):
            pltpu.sync_copy(gx_hbm.at[pl.ds(start, c)], gix.at[pl.ds(0, c)])
            pltpu.sync_copy(rc_hbm.at[pl.ds(start, c)], rcv.at[pl.ds(0, c)])

    plsc.subcore_barrier()

    for b in range(NBUF):
        pltpu.async_copy(a_hbm.at[gix.at[b]], bufs[b], gsems[b])

    def quad(i, _):
        k0 = NBUF * i
        for b in range(NBUF):
            pltpu.make_async_copy(a_hbm.at[gix.at[k0 + b]], bufs[b], gsems[b]).wait()
            pltpu.async_copy(bufs[b], agg_sh.at[rcv.at[k0 + b]], ssems[b], add=True)
        for b in range(NBUF):
            @pl.when(k0 + b + NBUF < cnt)
            def _(b=b):
                pltpu.make_async_copy(bufs[b], agg_sh.at[rcv.at[k0]], ssems[b]).wait()
                pltpu.async_copy(a_hbm.at[gix.at[k0 + b + NBUF]], bufs[b], gsems[b])
        return 0

    lax.fori_loop(0, cnt // NBUF, quad, 0)

    for b in range(NBUF):
        pltpu.make_async_copy(bufs[b], agg_sh.at[rcv.at[0]], ssems[b]).wait()

    plsc.subcore_barrier()

    r0 = sid * ROWS_PER_TILE
    pltpu.sync_copy(agg_sh.at[pl.ds(r0, ROWS_PER_TILE)],
                    out_hbm.at[cid, pl.ds(r0, ROWS_PER_TILE)])


def _finalize_body(parts_ref, hr_ref, nn_ref, w_ref, b_ref, out_ref):
    rdeg = jnp.sum(hr_ref[...], axis=0)
    nr = lax.rsqrt(jnp.maximum(rdeg, 1.0))
    nr_row = jnp.concatenate(
        [nr, jnp.zeros((N_PAD - N_NODES,), jnp.float32)]
    ).reshape(1, N_PAD)
    h = jnp.maximum(parts_ref[0] + parts_ref[1], 0.0)
    pooled = jnp.dot(nr_row, h, preferred_element_type=jnp.float32)
    cnt = jnp.maximum(nn_ref[0].astype(jnp.float32), 1.0)
    emb = pooled / cnt
    out_ref[...] = jnp.dot(emb, w_ref[...], preferred_element_type=jnp.float32) + b_ref[...]


_finalize = pl.pallas_call(
    _finalize_body,
    in_specs=[
        pl.BlockSpec(memory_space=pltpu.VMEM),
        pl.BlockSpec(memory_space=pltpu.VMEM),
        pl.BlockSpec(memory_space=pltpu.SMEM),
        pl.BlockSpec(memory_space=pltpu.VMEM),
        pl.BlockSpec(memory_space=pltpu.VMEM),
    ],
    out_shape=jax.ShapeDtypeStruct((1, HIDDEN), jnp.float32),
)


def kernel(nodes, senders, receivers, edge_types, n_node, kernels_0,
           dense_kernel, dense_bias):
    hist_s, hist_r, gidx = _make_degrees_and_gidx()(senders, receivers, edge_types)
    a_tab = _rel_matmul(hist_s.reshape(NW, N_NODES), nodes, kernels_0)
    zeros = jnp.zeros((ROWS_PER_TILE, HIDDEN), jnp.float32)
    parts = _make_edge_aggregate()(
        a_tab,
        gidx.reshape(NCHUNK, CHUNK),
        receivers.reshape(NCHUNK, CHUNK),
        zeros,
    )
    out = _finalize(parts, hist_r.reshape(NW, N_NODES), n_node, dense_kernel,
                    dense_bias.reshape(1, HIDDEN))
    return out.reshape(HIDDEN)


# speedup vs baseline: 1.0556x; 1.0100x over previous
import functools

import jax
import jax.numpy as jnp
from jax import lax
from jax.experimental import pallas as pl
from jax.experimental.pallas import tpu as pltpu
from jax.experimental.pallas import tpu_sc as plsc

N_NODES = 10000
N_EDGES = 320000
D_FEAT = 128
HIDDEN = 64
N_REL = 4

NC = 2
NS = 16
NW = NC * NS
L = 16

EPT = N_EDGES // NW
CHUNK = 128
NCHUNK = N_EDGES // CHUNK
N_PAD = 10240
ROWS_PER_TILE = N_PAD // NS


def _chunk_start(w):
    return (NCHUNK * w // NW) // 8 * 8


_COUNTS = sorted({
    (NCHUNK if w == NW - 1 else _chunk_start(w + 1)) - _chunk_start(w)
    for w in range(NW)
})
MAXC = max(_COUNTS)
NBUF = 4
assert all(c % NBUF == 0 for c in _COUNTS)

@functools.lru_cache(maxsize=None)
def _make_degrees_and_gidx():
    mesh = plsc.VectorSubcoreMesh(
        core_axis_name="c", subcore_axis_name="s", num_cores=NC, num_subcores=NS
    )
    return functools.partial(
        pl.kernel,
        out_type=(
            jax.ShapeDtypeStruct((NW, 1, N_NODES), jnp.float32),
            jax.ShapeDtypeStruct((NW, 1, N_NODES), jnp.float32),
            jax.ShapeDtypeStruct((N_EDGES,), jnp.int32),
        ),
        mesh=mesh,
        compiler_params=pltpu.CompilerParams(
            needs_layout_passes=False, use_tc_tiling_on_sc=False
        ),
        scratch_types=[
            pltpu.VMEM((N_NODES,), jnp.float32),
            pltpu.VMEM((N_NODES,), jnp.float32),
            pltpu.VMEM((EPT,), jnp.int32),
            pltpu.VMEM((EPT,), jnp.int32),
            pltpu.VMEM((EPT,), jnp.int32),
            pltpu.VMEM((EPT,), jnp.int32),
            pltpu.SemaphoreType.DMA,
        ],
    )(_degrees_and_gidx_body)


def _degrees_and_gidx_body(sen_hbm, rcv_hbm, et_hbm, hs_out, hr_out, gx_out,
                           hs, hr, sen, rcv, et, gix, ldsem):
    cid = lax.axis_index("c")
    sid = lax.axis_index("s")
    wid = sid * NC + cid
    base = wid * EPT

    pltpu.async_copy(sen_hbm.at[pl.ds(base, EPT)], sen, ldsem)
    pltpu.async_copy(rcv_hbm.at[pl.ds(base, EPT)], rcv, ldsem)
    pltpu.async_copy(et_hbm.at[pl.ds(base, EPT)], et, ldsem)

    def zero_body(j, _):
        z = jnp.zeros((L,), jnp.float32)
        hs[pl.ds(j * L, L)] = z
        hr[pl.ds(j * L, L)] = z
        return 0

    lax.fori_loop(0, N_NODES // L, zero_body, 0)

    pltpu.make_async_copy(sen_hbm.at[pl.ds(base, EPT)], sen, ldsem).wait()
    pltpu.make_async_copy(rcv_hbm.at[pl.ds(base, EPT)], rcv, ldsem).wait()
    pltpu.make_async_copy(et_hbm.at[pl.ds(base, EPT)], et, ldsem).wait()

    ones = jnp.ones((L,), jnp.float32)

    def body(k, _):
        s_v = sen[pl.ds(k * L, L)]
        r_v = rcv[pl.ds(k * L, L)]
        t_v = et[pl.ds(k * L, L)]
        plsc.addupdate_scatter(hs, [s_v], ones)
        plsc.addupdate_scatter(hr, [r_v], ones)
        gix[pl.ds(k * L, L)] = t_v * N_NODES + s_v
        return 0

    lax.fori_loop(0, EPT // L, body, 0)

    pltpu.sync_copy(gix, gx_out.at[pl.ds(base, EPT)])
    pltpu.sync_copy(hs, hs_out.at[wid, 0])
    pltpu.sync_copy(hr, hr_out.at[wid, 0])


BN = 1000


def _rel_matmul_body(hs_ref, nodes_ref, k_ref, out_ref):
    sdeg_col = lax.dot_general(
        hs_ref[...], jnp.ones((NW, 1), jnp.float32),
        (((0,), (0,)), ((), ())), preferred_element_type=jnp.float32)
    ns_col = lax.rsqrt(jnp.maximum(sdeg_col, 1.0))
    x = nodes_ref[...] * ns_col
    out_ref[...] = jnp.dot(x, k_ref[0], preferred_element_type=jnp.float32)


_rel_matmul = pl.pallas_call(
    _rel_matmul_body,
    grid=(N_REL,),
    in_specs=[
        pl.BlockSpec((NW, N_NODES), lambda r: (0, 0)),
        pl.BlockSpec((N_NODES, D_FEAT), lambda r: (0, 0)),
        pl.BlockSpec((1, D_FEAT, HIDDEN), lambda r: (r, 0, 0)),
    ],
    out_specs=pl.BlockSpec((N_NODES, HIDDEN), lambda r: (r, 0)),
    out_shape=jax.ShapeDtypeStruct((N_REL * N_NODES, HIDDEN), jnp.float32),
)


@functools.lru_cache(maxsize=None)
def _make_edge_aggregate():
    mesh = plsc.VectorSubcoreMesh(
        core_axis_name="c", subcore_axis_name="s", num_cores=NC, num_subcores=NS
    )
    return functools.partial(
        pl.kernel,
        out_type=jax.ShapeDtypeStruct((NC, N_PAD, HIDDEN), jnp.float32),
        mesh=mesh,
        compiler_params=pltpu.CompilerParams(
            needs_layout_passes=False, use_tc_tiling_on_sc=False
        ),
        scratch_types=[
            pltpu.VMEM_SHARED((N_PAD, HIDDEN), jnp.float32),
            pltpu.VMEM((MAXC, CHUNK), jnp.int32),
            pltpu.VMEM((MAXC, CHUNK), jnp.int32),
        ]
        + [pltpu.VMEM((CHUNK, HIDDEN), jnp.float32) for _ in range(NBUF)]
        + [pltpu.SemaphoreType.DMA for _ in range(2 * NBUF)],
    )(_edge_aggregate_body)


def _edge_aggregate_body(a_hbm, gx_hbm, rc_hbm, z_hbm, out_hbm,
                         agg_sh, gix, rcv, *rest):
    bufs = rest[:NBUF]
    gsems = rest[NBUF:2 * NBUF]
    ssems = rest[2 * NBUF:]
    cid = lax.axis_index("c")
    sid = lax.axis_index("s")
    wid = sid * NC + cid

    start = _chunk_start(wid)
    cnt = jnp.where(wid == NW - 1, NCHUNK, _chunk_start(wid + 1)) - start
    for c in _COUNTS:
        @pl.when(cnt == c)
        def _(c=c):
            pltpu.async_copy(gx_hbm.at[pl.ds(start, c)], gix.at[pl.ds(0, c)],
                             gsems[0])
            pltpu.async_copy(rc_hbm.at[pl.ds(start, c)], rcv.at[pl.ds(0, c)],
                             gsems[1])

    pltpu.sync_copy(z_hbm, agg_sh.at[pl.ds(sid * ROWS_PER_TILE, ROWS_PER_TILE)])

    for c in _COUNTS:
        @pl.when(cnt == c)
        def _(c=c):
            pltpu.make_async_copy(gx_hbm.at[pl.ds(start, c)],
                                  gix.at[pl.ds(0, c)], gsems[0]).wait()
            pltpu.make_async_copy(rc_hbm.at[pl.ds(start, c)],
                                  rcv.at[pl.ds(0, c)], gsems[1]).wait()

    plsc.subcore_barrier()

    for b in range(NBUF):
        pltpu.async_copy(a_hbm.at[gix.at[b]], bufs[b], gsems[b])

    def quad(i, _):
        k0 = NBUF * i
        for b in range(NBUF):
            pltpu.make_async_copy(a_hbm.at[gix.at[k0 + b]], bufs[b], gsems[b]).wait()
            pltpu.async_copy(bufs[b], agg_sh.at[rcv.at[k0 + b]], ssems[b], add=True)
        for b in range(NBUF):
            @pl.when(k0 + b + NBUF < cnt)
            def _(b=b):
                pltpu.make_async_copy(bufs[b], agg_sh.at[rcv.at[k0]], ssems[b]).wait()
                pltpu.async_copy(a_hbm.at[gix.at[k0 + b + NBUF]], bufs[b], gsems[b])
        return 0

    lax.fori_loop(0, cnt // NBUF, quad, 0)

    for b in range(NBUF):
        pltpu.make_async_copy(bufs[b], agg_sh.at[rcv.at[0]], ssems[b]).wait()

    plsc.subcore_barrier()

    r0 = sid * ROWS_PER_TILE
    pltpu.sync_copy(agg_sh.at[pl.ds(r0, ROWS_PER_TILE)],
                    out_hbm.at[cid, pl.ds(r0, ROWS_PER_TILE)])


def _finalize_body(parts_ref, hr_ref, nn_ref, w_ref, b_ref, out_ref):
    rdeg = jnp.sum(hr_ref[...], axis=0)
    nr = lax.rsqrt(jnp.maximum(rdeg, 1.0))
    nr_row = jnp.concatenate(
        [nr, jnp.zeros((N_PAD - N_NODES,), jnp.float32)]
    ).reshape(1, N_PAD)
    h = jnp.maximum(parts_ref[0] + parts_ref[1], 0.0)
    pooled = jnp.dot(nr_row, h, preferred_element_type=jnp.float32)
    cnt = jnp.maximum(nn_ref[0].astype(jnp.float32), 1.0)
    emb = pooled / cnt
    out_ref[...] = jnp.dot(emb, w_ref[...], preferred_element_type=jnp.float32) + b_ref[...]


_finalize = pl.pallas_call(
    _finalize_body,
    in_specs=[
        pl.BlockSpec(memory_space=pltpu.VMEM),
        pl.BlockSpec(memory_space=pltpu.VMEM),
        pl.BlockSpec(memory_space=pltpu.SMEM),
        pl.BlockSpec(memory_space=pltpu.VMEM),
        pl.BlockSpec(memory_space=pltpu.VMEM),
    ],
    out_shape=jax.ShapeDtypeStruct((1, HIDDEN), jnp.float32),
)


def kernel(nodes, senders, receivers, edge_types, n_node, kernels_0,
           dense_kernel, dense_bias):
    hist_s, hist_r, gidx = _make_degrees_and_gidx()(senders, receivers, edge_types)
    a_tab = _rel_matmul(hist_s.reshape(NW, N_NODES), nodes, kernels_0)
    zeros = jnp.zeros((ROWS_PER_TILE, HIDDEN), jnp.float32)
    parts = _make_edge_aggregate()(
        a_tab,
        gidx.reshape(NCHUNK, CHUNK),
        receivers.reshape(NCHUNK, CHUNK),
        zeros,
    )
    out = _finalize(parts, hist_r.reshape(NW, N_NODES), n_node, dense_kernel,
                    dense_bias.reshape(1, HIDDEN))
    return out.reshape(HIDDEN)


# speedup vs baseline: 1.0596x; 1.0038x over previous
import functools

import jax
import jax.numpy as jnp
from jax import lax
from jax.experimental import pallas as pl
from jax.experimental.pallas import tpu as pltpu
from jax.experimental.pallas import tpu_sc as plsc

N_NODES = 10000
N_EDGES = 320000
D_FEAT = 128
HIDDEN = 64
N_REL = 4

NC = 2
NS = 16
NW = NC * NS
L = 16

EPT = N_EDGES // NW
CHUNK = 128
NCHUNK = N_EDGES // CHUNK
N_PAD = 10240
ROWS_PER_TILE = N_PAD // NS


def _chunk_start(w):
    return (NCHUNK * w // NW) // 8 * 8


_COUNTS = sorted({
    (NCHUNK if w == NW - 1 else _chunk_start(w + 1)) - _chunk_start(w)
    for w in range(NW)
})
MAXC = max(_COUNTS)
QUAD = 4
NBUF = 8
assert all(c % QUAD == 0 for c in _COUNTS)
assert all(c >= 2 * QUAD for c in _COUNTS)

@functools.lru_cache(maxsize=None)
def _make_degrees_and_gidx():
    mesh = plsc.VectorSubcoreMesh(
        core_axis_name="c", subcore_axis_name="s", num_cores=NC, num_subcores=NS
    )
    return functools.partial(
        pl.kernel,
        out_type=(
            jax.ShapeDtypeStruct((NW, 1, N_NODES), jnp.float32),
            jax.ShapeDtypeStruct((NW, 1, N_NODES), jnp.float32),
            jax.ShapeDtypeStruct((N_EDGES,), jnp.int32),
        ),
        mesh=mesh,
        compiler_params=pltpu.CompilerParams(
            needs_layout_passes=False, use_tc_tiling_on_sc=False
        ),
        scratch_types=[
            pltpu.VMEM((N_NODES,), jnp.float32),
            pltpu.VMEM((N_NODES,), jnp.float32),
            pltpu.VMEM((EPT,), jnp.int32),
            pltpu.VMEM((EPT,), jnp.int32),
            pltpu.VMEM((EPT,), jnp.int32),
            pltpu.VMEM((EPT,), jnp.int32),
            pltpu.SemaphoreType.DMA,
        ],
    )(_degrees_and_gidx_body)


def _degrees_and_gidx_body(sen_hbm, rcv_hbm, et_hbm, hs_out, hr_out, gx_out,
                           hs, hr, sen, rcv, et, gix, ldsem):
    cid = lax.axis_index("c")
    sid = lax.axis_index("s")
    wid = sid * NC + cid
    base = wid * EPT

    pltpu.async_copy(sen_hbm.at[pl.ds(base, EPT)], sen, ldsem)
    pltpu.async_copy(rcv_hbm.at[pl.ds(base, EPT)], rcv, ldsem)
    pltpu.async_copy(et_hbm.at[pl.ds(base, EPT)], et, ldsem)

    def zero_body(j, _):
        z = jnp.zeros((L,), jnp.float32)
        hs[pl.ds(j * L, L)] = z
        hr[pl.ds(j * L, L)] = z
        return 0

    lax.fori_loop(0, N_NODES // L, zero_body, 0)

    pltpu.make_async_copy(sen_hbm.at[pl.ds(base, EPT)], sen, ldsem).wait()
    pltpu.make_async_copy(rcv_hbm.at[pl.ds(base, EPT)], rcv, ldsem).wait()
    pltpu.make_async_copy(et_hbm.at[pl.ds(base, EPT)], et, ldsem).wait()

    ones = jnp.ones((L,), jnp.float32)

    def body(k, _):
        s_v = sen[pl.ds(k * L, L)]
        r_v = rcv[pl.ds(k * L, L)]
        t_v = et[pl.ds(k * L, L)]
        plsc.addupdate_scatter(hs, [s_v], ones)
        plsc.addupdate_scatter(hr, [r_v], ones)
        gix[pl.ds(k * L, L)] = t_v * N_NODES + s_v
        return 0

    lax.fori_loop(0, EPT // L, body, 0)

    pltpu.sync_copy(gix, gx_out.at[pl.ds(base, EPT)])
    pltpu.sync_copy(hs, hs_out.at[wid, 0])
    pltpu.sync_copy(hr, hr_out.at[wid, 0])


BN = 1000


def _rel_matmul_body(hs_ref, nodes_ref, k_ref, out_ref):
    sdeg_col = lax.dot_general(
        hs_ref[...], jnp.ones((NW, 1), jnp.float32),
        (((0,), (0,)), ((), ())), preferred_element_type=jnp.float32)
    ns_col = lax.rsqrt(jnp.maximum(sdeg_col, 1.0))
    x = nodes_ref[...] * ns_col
    out_ref[...] = jnp.dot(x, k_ref[0], preferred_element_type=jnp.float32)


_rel_matmul = pl.pallas_call(
    _rel_matmul_body,
    grid=(N_REL,),
    in_specs=[
        pl.BlockSpec((NW, N_NODES), lambda r: (0, 0)),
        pl.BlockSpec((N_NODES, D_FEAT), lambda r: (0, 0)),
        pl.BlockSpec((1, D_FEAT, HIDDEN), lambda r: (r, 0, 0)),
    ],
    out_specs=pl.BlockSpec((N_NODES, HIDDEN), lambda r: (r, 0)),
    out_shape=jax.ShapeDtypeStruct((N_REL * N_NODES, HIDDEN), jnp.float32),
)


@functools.lru_cache(maxsize=None)
def _make_edge_aggregate():
    mesh = plsc.VectorSubcoreMesh(
        core_axis_name="c", subcore_axis_name="s", num_cores=NC, num_subcores=NS
    )
    return functools.partial(
        pl.kernel,
        out_type=jax.ShapeDtypeStruct((NC, N_PAD, HIDDEN), jnp.float32),
        mesh=mesh,
        compiler_params=pltpu.CompilerParams(
            needs_layout_passes=False, use_tc_tiling_on_sc=False
        ),
        scratch_types=[
            pltpu.VMEM_SHARED((N_PAD, HIDDEN), jnp.float32),
            pltpu.VMEM((MAXC, CHUNK), jnp.int32),
            pltpu.VMEM((MAXC, CHUNK), jnp.int32),
        ]
        + [pltpu.VMEM((CHUNK, HIDDEN), jnp.float32) for _ in range(NBUF)]
        + [pltpu.SemaphoreType.DMA for _ in range(2 * NBUF)],
    )(_edge_aggregate_body)


def _edge_aggregate_body(a_hbm, gx_hbm, rc_hbm, z_hbm, out_hbm,
                         agg_sh, gix, rcv, *rest):
    bufs = rest[:NBUF]
    gsems = rest[NBUF:2 * NBUF]
    ssems = rest[2 * NBUF:]
    cid = lax.axis_index("c")
    sid = lax.axis_index("s")
    wid = sid * NC + cid

    start = _chunk_start(wid)
    cnt = jnp.where(wid == NW - 1, NCHUNK, _chunk_start(wid + 1)) - start
    for c in _COUNTS:
        @pl.when(cnt == c)
        def _(c=c):
            pltpu.async_copy(gx_hbm.at[pl.ds(start, c)], gix.at[pl.ds(0, c)],
                             gsems[0])
            pltpu.async_copy(rc_hbm.at[pl.ds(start, c)], rcv.at[pl.ds(0, c)],
                             gsems[1])

    pltpu.sync_copy(z_hbm, agg_sh.at[pl.ds(sid * ROWS_PER_TILE, ROWS_PER_TILE)])

    for c in _COUNTS:
        @pl.when(cnt == c)
        def _(c=c):
            pltpu.make_async_copy(gx_hbm.at[pl.ds(start, c)],
                                  gix.at[pl.ds(0, c)], gsems[0]).wait()
            pltpu.make_async_copy(rc_hbm.at[pl.ds(start, c)],
                                  rcv.at[pl.ds(0, c)], gsems[1]).wait()

    plsc.subcore_barrier()

    for b in range(QUAD):
        pltpu.async_copy(a_hbm.at[gix.at[b]], bufs[b], gsems[b])

    def quad(i, _):
        k0 = QUAD * i
        even = (i % 2) == 0
        for bank0 in (0, QUAD):
            @pl.when(even if bank0 == 0 else jnp.logical_not(even))
            def _(bank0=bank0):
                for b in range(QUAD):
                    buf = bank0 + b
                    pltpu.make_async_copy(
                        a_hbm.at[gix.at[k0 + b]], bufs[buf], gsems[buf]).wait()
                    pltpu.async_copy(
                        bufs[buf], agg_sh.at[rcv.at[k0 + b]], ssems[buf], add=True)
                for b in range(QUAD):
                    buf = (bank0 + QUAD + b) % NBUF
                    k_next = k0 + QUAD + b

                    @pl.when(k_next < cnt)
                    def _(buf=buf, k_next=k_next):
                        @pl.when(i > 0)
                        def _():
                            pltpu.make_async_copy(
                                bufs[buf], agg_sh.at[rcv.at[0]], ssems[buf]).wait()
                        pltpu.async_copy(
                            a_hbm.at[gix.at[k_next]], bufs[buf], gsems[buf])
        return 0

    lax.fori_loop(0, cnt // QUAD, quad, 0)

    for b in range(NBUF):
        pltpu.make_async_copy(bufs[b], agg_sh.at[rcv.at[0]], ssems[b]).wait()

    plsc.subcore_barrier()

    r0 = sid * ROWS_PER_TILE
    pltpu.sync_copy(agg_sh.at[pl.ds(r0, ROWS_PER_TILE)],
                    out_hbm.at[cid, pl.ds(r0, ROWS_PER_TILE)])


def _finalize_body(parts_ref, hr_ref, nn_ref, w_ref, b_ref, out_ref):
    rdeg = jnp.sum(hr_ref[...], axis=0)
    nr = lax.rsqrt(jnp.maximum(rdeg, 1.0))
    nr_row = jnp.concatenate(
        [nr, jnp.zeros((N_PAD - N_NODES,), jnp.float32)]
    ).reshape(1, N_PAD)
    h = jnp.maximum(parts_ref[0] + parts_ref[1], 0.0)
    pooled = jnp.dot(nr_row, h, preferred_element_type=jnp.float32)
    cnt = jnp.maximum(nn_ref[0].astype(jnp.float32), 1.0)
    emb = pooled / cnt
    out_ref[...] = jnp.dot(emb, w_ref[...], preferred_element_type=jnp.float32) + b_ref[...]


_finalize = pl.pallas_call(
    _finalize_body,
    in_specs=[
        pl.BlockSpec(memory_space=pltpu.VMEM),
        pl.BlockSpec(memory_space=pltpu.VMEM),
        pl.BlockSpec(memory_space=pltpu.SMEM),
        pl.BlockSpec(memory_space=pltpu.VMEM),
        pl.BlockSpec(memory_space=pltpu.VMEM),
    ],
    out_shape=jax.ShapeDtypeStruct((1, HIDDEN), jnp.float32),
)


def kernel(nodes, senders, receivers, edge_types, n_node, kernels_0,
           dense_kernel, dense_bias):
    hist_s, hist_r, gidx = _make_degrees_and_gidx()(senders, receivers, edge_types)
    a_tab = _rel_matmul(hist_s.reshape(NW, N_NODES), nodes, kernels_0)
    zeros = jnp.zeros((ROWS_PER_TILE, HIDDEN), jnp.float32)
    parts = _make_edge_aggregate()(
        a_tab,
        gidx.reshape(NCHUNK, CHUNK),
        receivers.reshape(NCHUNK, CHUNK),
        zeros,
    )
    out = _finalize(parts, hist_r.reshape(NW, N_NODES), n_node, dense_kernel,
                    dense_bias.reshape(1, HIDDEN))
    return out.reshape(HIDDEN)
